# baseline (speedup 1.0000x reference)
import functools

import jax
import jax.numpy as jnp
from jax import lax
from jax.experimental import pallas as pl
from jax.experimental.pallas import tpu as pltpu
from jax.experimental.pallas import tpu_sc as plsc

D_MODEL = 64
SCALE = 8.0

LANES = 128
STEPS = 4
CHUNK = LANES * STEPS


@functools.lru_cache(maxsize=None)
def _make_gather(n_idx: int):
  info = plsc.get_sparse_core_info()
  nc, ns = info.num_cores, info.num_subcores
  nw = nc * ns
  rows_total = n_idx // LANES
  rows_per_w = rows_total // nw
  chunks = rows_per_w // STEPS
  mesh = plsc.VectorSubcoreMesh(core_axis_name="c", subcore_axis_name="s")

  @functools.partial(
      pl.kernel,
      mesh=mesh,
      compiler_params=pltpu.CompilerParams(use_tc_tiling_on_sc=False),
      out_type=jax.ShapeDtypeStruct((n_idx, D_MODEL), jnp.float32),
      scratch_types=[
          pltpu.VMEM((STEPS, LANES), jnp.int32),
          pltpu.VMEM((CHUNK, D_MODEL), jnp.float32),
          pltpu.SemaphoreType.DMA,
      ],
  )
  def k(idx_hbm, table_hbm, out_hbm, idx_v, rows_v, sem):
    wid = lax.axis_index("s") * nc + lax.axis_index("c")
    row0 = wid * rows_per_w

    def chunk_body(c, carry):
      r0 = row0 + c * STEPS
      pltpu.sync_copy(idx_hbm.at[pl.ds(r0, STEPS)], idx_v)
      copies = [
          pltpu.async_copy(table_hbm.at[idx_v.at[j]],
                           rows_v.at[pl.ds(j * LANES, LANES)], sem)
          for j in range(STEPS)
      ]
      for cp in copies:
        cp.wait()

      def scale_body(t, inner):
        for kk in range(D_MODEL // 16):
          sl = pl.ds(kk * 16, 16)
          rows_v[t, sl] = rows_v[t, sl] * SCALE
        return inner

      lax.fori_loop(0, CHUNK, scale_body, 0)
      pltpu.sync_copy(rows_v, out_hbm.at[pl.ds(r0 * LANES, CHUNK)])
      return carry

    lax.fori_loop(0, chunks, chunk_body, 0)

  return k


def kernel(x, table):
  b0, b1 = x.shape
  n = b0 * b1
  idx = x.reshape(n // LANES, LANES).astype(jnp.int32)
  out = _make_gather(n)(idx, table)
  return out.reshape(b0, b1, D_MODEL)


# speedup vs baseline: 1.1234x; 1.1234x over previous
import functools

import jax
import jax.numpy as jnp
from jax import lax
from jax.experimental import pallas as pl
from jax.experimental.pallas import tpu as pltpu
from jax.experimental.pallas import tpu_sc as plsc

D_MODEL = 64
SCALE = 8.0

LANES = 128
STEPS = 4
CHUNK = LANES * STEPS
NBUF = 2


@functools.lru_cache(maxsize=None)
def _make_gather(n_idx: int):
  info = plsc.get_sparse_core_info()
  nc, ns = info.num_cores, info.num_subcores
  nw = nc * ns
  rows_total = n_idx // LANES
  rows_per_w = rows_total // nw
  chunks = rows_per_w // STEPS
  mesh = plsc.VectorSubcoreMesh(core_axis_name="c", subcore_axis_name="s")

  @functools.partial(
      pl.kernel,
      mesh=mesh,
      compiler_params=pltpu.CompilerParams(use_tc_tiling_on_sc=False),
      out_type=jax.ShapeDtypeStruct((n_idx, D_MODEL), jnp.float32),
      scratch_types=[
          pltpu.VMEM((NBUF, STEPS, LANES), jnp.int32),
          pltpu.VMEM((NBUF, CHUNK, D_MODEL), jnp.float32),
          pltpu.SemaphoreType.DMA,
          pltpu.SemaphoreType.DMA,
          pltpu.SemaphoreType.DMA,
          pltpu.SemaphoreType.DMA,
      ],
  )
  def k(idx_hbm, table_hbm, out_hbm, idx_v, rows_v, sg0, sg1, sw0, sw1):
    wid = lax.axis_index("s") * nc + lax.axis_index("c")
    row0 = wid * rows_per_w
    sem_g = (sg0, sg1)
    sem_w = (sw0, sw1)

    def gather_copies(b):
      return [
          pltpu.make_async_copy(table_hbm.at[idx_v.at[b, j]],
                                rows_v.at[b, pl.ds(j * LANES, LANES)],
                                sem_g[b])
          for j in range(STEPS)
      ]

    def fire(c, b):
      pltpu.sync_copy(idx_hbm.at[pl.ds(row0 + c * STEPS, STEPS)],
                      idx_v.at[b])
      for cp in gather_copies(b):
        cp.start()

    def drain_gathers(b):
      for cp in gather_copies(b):
        cp.wait()

    def out_copy(c, b):
      return pltpu.make_async_copy(
          rows_v.at[b],
          out_hbm.at[pl.ds((row0 + c * STEPS) * LANES, CHUNK)],
          sem_w[b])

    def put(c, b):
      @pl.loop(0, CHUNK, unroll=8)
      def _(t):
        for kk in range(D_MODEL // 16):
          sl = pl.ds(kk * 16, 16)
          rows_v[b, t, sl] = rows_v[b, t, sl] * SCALE

      out_copy(c, b).start()

    fire(0, 0)

    @pl.loop(0, chunks, step=NBUF)
    def _(c0):
      for boff in range(NBUF):
        c = c0 + boff
        b = boff
        nb = 1 - boff

        @pl.when(c >= 1)
        def _():
          out_copy(c - 1, nb).wait()

        @pl.when(c + 1 < chunks)
        def _():
          fire(c + 1, nb)

        drain_gathers(b)
        put(c, b)

    out_copy(chunks - 1, (chunks - 1) % NBUF).wait()

  return k


def kernel(x, table):
  b0, b1 = x.shape
  n = b0 * b1
  idx = x.reshape(n // LANES, LANES).astype(jnp.int32)
  out = _make_gather(n)(idx, table)
  return out.reshape(b0, b1, D_MODEL)
